# baseline (speedup 1.0000x reference)
import functools

import jax
import jax.numpy as jnp
from jax import lax
from jax.experimental import pallas as pl
from jax.experimental.pallas import tpu as pltpu
from jax.experimental.pallas import tpu_sc as plsc

N_ROWS = 100000
N_COLS = 128
NUM_SEG = 512

NUM_CORES = 2
NUM_SUBCORES = 16
COLS_PER_CORE = N_COLS // NUM_CORES
SEG_PER_TILE = NUM_SEG // NUM_SUBCORES

MB = 8
MPT = 800
TOT_MB = NUM_SUBCORES * MPT
DUMMY = NUM_SEG
ACC_ROWS = NUM_SEG + 8
BB = 16

_TC_BLOCK = 4000
_TC_MBLOCK = _TC_BLOCK // MB


def _fused_tc_body(x_ref, o_ref, m_ref, ms_ref, msem):
    i = pl.program_id(0)
    xb = x_ref[...]
    o_ref[...] = xb
    ms_ref[...] = xb.reshape(_TC_MBLOCK, MB, N_COLS).sum(axis=1)
    pltpu.make_async_copy(
        ms_ref, m_ref.at[pl.ds(i * _TC_MBLOCK, _TC_MBLOCK), :], msem
    ).start()
    pltpu.make_async_copy(
        ms_ref, m_ref.at[pl.ds(i * _TC_MBLOCK, _TC_MBLOCK), :], msem
    ).wait()


def _fused_tc(x):
    return pl.pallas_call(
        _fused_tc_body,
        grid=(N_ROWS // _TC_BLOCK,),
        in_specs=[pl.BlockSpec((_TC_BLOCK, N_COLS), lambda i: (i, 0))],
        out_specs=[
            pl.BlockSpec((_TC_BLOCK, N_COLS), lambda i: (i, 0)),
            pl.BlockSpec(memory_space=pl.ANY),
        ],
        out_shape=[
            jax.ShapeDtypeStruct((N_ROWS, N_COLS), jnp.float32),
            jax.ShapeDtypeStruct((TOT_MB, N_COLS), jnp.float32),
        ],
        scratch_shapes=[
            pltpu.VMEM((_TC_MBLOCK, N_COLS), jnp.float32),
            pltpu.SemaphoreType.DMA,
        ],
    )(x)


@functools.partial(
    pl.kernel,
    mesh=plsc.VectorSubcoreMesh(core_axis_name="c", subcore_axis_name="s"),
    out_type=jax.ShapeDtypeStruct((NUM_SEG, N_COLS), jnp.float32),
    scratch_types=[
        pltpu.VMEM((MPT, MB), jnp.int32),
        pltpu.VMEM((MPT, COLS_PER_CORE), jnp.float32),
        pltpu.VMEM((MPT,), jnp.int32),
        pltpu.VMEM((MPT + 16,), jnp.int32),
        pltpu.VMEM((BB * MB,), jnp.int32),
        pltpu.VMEM((BB * MB,), jnp.int32),
        pltpu.VMEM((BB * MB, N_COLS), jnp.float32),
        pltpu.VMEM((BB * MB, COLS_PER_CORE), jnp.float32),
        pltpu.VMEM((SEG_PER_TILE, COLS_PER_CORE), jnp.float32),
        pltpu.VMEM_SHARED((ACC_ROWS, COLS_PER_CORE), jnp.float32),
        pltpu.SemaphoreType.DMA,
        pltpu.SemaphoreType.DMA,
        pltpu.SemaphoreType.DMA,
    ],
    compiler_params=pltpu.CompilerParams(use_tc_tiling_on_sc=False,
                                        needs_layout_passes=False),
)
def _segment_sum_sc(x_hbm, ids_hbm, m_hbm, out_hbm, ids_v, m_v, idx_v,
                    bnd_v, brow_v, bids_v, bx_v, bxh_v, stage_v, acc_sh,
                    sem_m, sem_b, sem_sc):
    c = lax.axis_index("c")
    s = lax.axis_index("s")
    col0 = c * COLS_PER_CORE

    zero16 = jnp.zeros((16,), jnp.float32)
    for r in range(SEG_PER_TILE):
        for k in range(COLS_PER_CORE // 16):
            stage_v[r, pl.ds(k * 16, 16)] = zero16
    pltpu.sync_copy(stage_v, acc_sh.at[pl.ds(s * SEG_PER_TILE, SEG_PER_TILE)])

    pltpu.sync_copy(ids_hbm.at[s], ids_v)
    mcopy = pltpu.async_copy(
        m_hbm.at[pl.ds(s * MPT, MPT), pl.ds(col0, COLS_PER_CORE)], m_v, sem_m)

    iota = lax.iota(jnp.int32, 16)
    izero = jnp.zeros((16,), jnp.int32)
    nb = jnp.int32(0)
    for g in range(MPT // 16):
        rowv = izero + g * 16 + iota
        head = plsc.load_gather(ids_v, [rowv, izero])
        tail = plsc.load_gather(ids_v, [rowv, izero + (MB - 1)])
        pure = head == tail
        idx_v[pl.ds(g * 16, 16)] = jnp.where(pure, head, DUMMY)
        impure = jnp.logical_not(pure)
        plsc.store_compressed(bnd_v.at[pl.ds(nb, 16)], rowv, mask=impure)
        nb = nb + plsc.all_reduce_population_count(impure)[0]

    plsc.subcore_barrier()

    mcopy.wait()
    pltpu.async_copy(m_v, acc_sh.at[idx_v], sem_sc, add=True).wait()

    nbat = (nb + (BB - 1)) // BB

    def bbody(kb, carry):
        nbs = izero + nb
        for m in range(BB // 2):
            p0 = kb * BB + 2 * m
            j0 = plsc.load_gather(bnd_v, [izero + p0])
            j1 = plsc.load_gather(bnd_v, [izero + p0 + 1])
            v0 = (izero + p0) < nbs
            v1 = (izero + p0 + 1) < nbs
            j0 = jnp.where(v0, j0, 0)
            j1 = jnp.where(v1, j1, 0)
            lo = iota < 8
            jmix = jnp.where(lo, j0, j1)
            vmix = jnp.where(lo, v0, v1)
            rofs = jnp.where(lo, iota, iota - 8)
            brow_v[pl.ds(16 * m, 16)] = (s * MPT + jmix) * MB + rofs
            segv = plsc.load_gather(ids_v, [jmix, rofs])
            bids_v[pl.ds(16 * m, 16)] = jnp.where(vmix, segv, DUMMY)
        pltpu.async_copy(x_hbm.at[brow_v], bx_v, sem_b).wait()

        def cbody(r, cc):
            for k in range(COLS_PER_CORE // 16):
                bxh_v[r, pl.ds(k * 16, 16)] = bx_v[r, pl.ds(col0 + k * 16, 16)]
            return cc

        lax.fori_loop(0, BB * MB, cbody, 0)
        pltpu.async_copy(bxh_v, acc_sh.at[bids_v], sem_sc, add=True).wait()
        return carry

    lax.fori_loop(0, nbat, bbody, 0)
    plsc.subcore_barrier()

    pltpu.sync_copy(
        acc_sh.at[pl.ds(s * SEG_PER_TILE, SEG_PER_TILE)],
        out_hbm.at[pl.ds(s * SEG_PER_TILE, SEG_PER_TILE),
                   pl.ds(col0, COLS_PER_CORE)],
    )


def kernel(x, batch):
    ids = batch.astype(jnp.int32)
    ids_p = jnp.concatenate(
        [ids, jnp.full((TOT_MB * MB - N_ROWS,), DUMMY, jnp.int32)]
    ).reshape(NUM_SUBCORES, MPT, MB)
    node_embedding, m = _fused_tc(x)
    graph_embedding = _segment_sum_sc(x, ids_p, m)
    return (node_embedding, graph_embedding)
